# Initial kernel scaffold; baseline (speedup 1.0000x reference)
import jax
import jax.numpy as jnp
from jax.experimental import pallas as pl


def kernel(x_paper, x_author, ei_cites, ei_writes, ei_rev, params):
    raise NotImplementedError("write your pallas kernel here")


# speedup vs baseline: 4.1116x; 4.1116x over previous
import functools
import numpy as np
import jax
import jax.numpy as jnp
from jax.experimental import pallas as pl

NODE_TYPES = ('paper', 'author')
EDGE_TYPES = (('paper', 'cites', 'paper'), ('author', 'writes', 'paper'), ('paper', 'rev_writes', 'author'))
D = 128
HEADS = 8
DH = D // HEADS
POOL = 8
OUT = 16

_BM = 1000


def _mm_kernel(x_ref, w_ref, b_ref, o_ref):
    acc = jnp.dot(x_ref[...], w_ref[...], preferred_element_type=jnp.float32)
    o_ref[...] = acc + b_ref[...]


def _mm(x, w, b):
    M, K = x.shape
    N = w.shape[1]
    bm = _BM if M % _BM == 0 else M
    grid = (M // bm,)
    return pl.pallas_call(
        _mm_kernel,
        grid=grid,
        in_specs=[
            pl.BlockSpec((bm, K), lambda i: (i, 0)),
            pl.BlockSpec((K, N), lambda i: (0, 0)),
            pl.BlockSpec((1, N), lambda i: (0, 0)),
        ],
        out_specs=pl.BlockSpec((bm, N), lambda i: (i, 0)),
        out_shape=jax.ShapeDtypeStruct((M, N), jnp.float32),
    )(x, w, b.reshape(1, N))


def _out_proj_kernel(agg_ref, w_ref, b_ref, res_ref, sk_ref, o_ref):
    a = jax.nn.gelu(agg_ref[...])
    o = jnp.dot(a, w_ref[...], preferred_element_type=jnp.float32) + b_ref[...]
    sk = sk_ref[...]
    o_ref[...] = sk * o + (1.0 - sk) * res_ref[...]


def _out_proj(agg, w, b, res, sk):
    M, K = agg.shape
    N = w.shape[1]
    bm = _BM if M % _BM == 0 else M
    grid = (M // bm,)
    sk_row = jnp.broadcast_to(sk.reshape(1, 1), (1, N))
    return pl.pallas_call(
        _out_proj_kernel,
        grid=grid,
        in_specs=[
            pl.BlockSpec((bm, K), lambda i: (i, 0)),
            pl.BlockSpec((K, N), lambda i: (0, 0)),
            pl.BlockSpec((1, N), lambda i: (0, 0)),
            pl.BlockSpec((bm, N), lambda i: (i, 0)),
            pl.BlockSpec((1, N), lambda i: (0, 0)),
        ],
        out_specs=pl.BlockSpec((bm, N), lambda i: (i, 0)),
        out_shape=jax.ShapeDtypeStruct((M, N), jnp.float32),
    )(agg, w, b.reshape(1, N), res, sk_row)


def _ln_gelu_kernel(x_ref, w_ref, b_ref, o_ref):
    x = x_ref[...]
    mu = x.mean(-1, keepdims=True)
    var = ((x - mu) ** 2).mean(-1, keepdims=True)
    y = (x - mu) / jnp.sqrt(var + 1e-5) * w_ref[...] + b_ref[...]
    o_ref[...] = jax.nn.gelu(y)


def _ln_gelu(x, w, b):
    M, N = x.shape
    bm = _BM if M % _BM == 0 else M
    grid = (M // bm,)
    return pl.pallas_call(
        _ln_gelu_kernel,
        grid=grid,
        in_specs=[
            pl.BlockSpec((bm, N), lambda i: (i, 0)),
            pl.BlockSpec((1, N), lambda i: (0, 0)),
            pl.BlockSpec((1, N), lambda i: (0, 0)),
        ],
        out_specs=pl.BlockSpec((bm, N), lambda i: (i, 0)),
        out_shape=jax.ShapeDtypeStruct((M, N), jnp.float32),
    )(x, w.reshape(1, N), b.reshape(1, N))


def _mlp_kernel(h_ref, w1_ref, b1_ref, w2_ref, b2_ref, w3_ref, b3_ref, w4_ref, b4_ref, o_ref):
    h = h_ref[...]
    h = jax.nn.gelu(jnp.dot(h, w1_ref[...], preferred_element_type=jnp.float32) + b1_ref[...])
    h = jax.nn.gelu(jnp.dot(h, w2_ref[...], preferred_element_type=jnp.float32) + b2_ref[...])
    h = jax.nn.gelu(jnp.dot(h, w3_ref[...], preferred_element_type=jnp.float32) + b3_ref[...])
    h = jnp.dot(h, w4_ref[...], preferred_element_type=jnp.float32) + b4_ref[...]
    o_ref[...] = jnp.nan_to_num(h)


def _mlp_head(h0, mp):
    out = pl.pallas_call(
        _mlp_kernel,
        out_shape=jax.ShapeDtypeStruct((1, OUT), jnp.float32),
    )(h0,
      mp['w1'], mp['b1'].reshape(1, -1),
      mp['w2'], mp['b2'].reshape(1, -1),
      mp['w3'], mp['b3'].reshape(1, -1),
      mp['w4'], mp['b4'].reshape(1, -1))
    return out.reshape(OUT)


def _segment_softmax(logits, seg, num_segments):
    m = jax.ops.segment_max(logits, seg, num_segments=num_segments)
    m = jnp.where(jnp.isfinite(m), m, 0.0)
    e = jnp.exp(logits - m[seg])
    s = jax.ops.segment_sum(e, seg, num_segments=num_segments)
    return e / (s[seg] + 1e-16)


def _hgt_conv(x_dict, ei_list, lp, bd_k, bd_v):
    k = {}
    q = {}
    v = {}
    for nt in NODE_TYPES:
        kqv = _mm(x_dict[nt], lp['kqv_w'][nt], lp['kqv_b'][nt])
        k[nt] = kqv[:, :D]
        q[nt] = kqv[:, D:2 * D]
        v[nt] = kqv[:, 2 * D:]
    offsets = {}
    off = 0
    for nt in NODE_TYPES:
        offsets[nt] = off
        off += x_dict[nt].shape[0]
    n_total = off
    alphas = []
    msgs = []
    dsts = []
    zero_b = jnp.zeros((D,), jnp.float32)
    for i, (src_t, rel, dst_t) in enumerate(EDGE_TYPES):
        ei = ei_list[i]
        src = ei[0]
        dst = ei[1]
        k_rel = _mm(k[src_t], bd_k[i], zero_b)
        v_rel = _mm(v[src_t], bd_v[i], zero_b)
        k_e = k_rel[src]
        v_e = v_rel[src]
        q_e = q[dst_t][dst]
        a = (q_e.reshape(-1, HEADS, DH) * k_e.reshape(-1, HEADS, DH)).sum(-1)
        a = a * lp['p_rel'][i] / np.sqrt(DH)
        alphas.append(a)
        msgs.append(v_e)
        dsts.append(dst + offsets[dst_t])
    alpha = jnp.concatenate(alphas, 0)
    msg = jnp.concatenate(msgs, 0)
    dst_all = jnp.concatenate(dsts, 0)
    attn = _segment_softmax(alpha, dst_all, n_total)
    attn_flat = jnp.repeat(attn, DH, axis=1)
    agg = jax.ops.segment_sum(msg * attn_flat, dst_all, num_segments=n_total)
    out = {}
    for nt in NODE_TYPES:
        o = agg[offsets[nt]:offsets[nt] + x_dict[nt].shape[0]]
        out[nt] = _out_proj(o, lp['out_w'][nt], lp['out_b'][nt], x_dict[nt], jax.nn.sigmoid(lp['skip'][nt]))
    return out


def _block_diag(rel):
    eye = jnp.eye(HEADS, dtype=jnp.float32)
    m = jnp.einsum('hde,hg->hdge', rel, eye)
    return m.reshape(D, D)


def _gat_score(x, src, dst, gp):
    n = x.shape[0]
    loop = jnp.arange(n, dtype=src.dtype)
    s = jnp.concatenate([src, loop])
    t = jnp.concatenate([dst, loop])
    wp = jnp.pad(gp['w'], ((0, 0), (0, 127)))
    h = _mm(x, wp, jnp.zeros((128,), jnp.float32))[:, 0]
    a = jax.nn.leaky_relu(gp['att_src'] * h[s] + gp['att_dst'] * h[t], 0.2)
    attn = _segment_softmax(a, t, n)
    out = jax.ops.segment_sum(h[s] * attn, t, num_segments=n)
    return out + gp['b']


def kernel(x_paper, x_author, ei_cites, ei_weights, ei_rev, params):
    ei_writes = ei_weights
    x = {'paper': x_paper, 'author': x_author}
    ei_list = [ei_cites, ei_writes, ei_rev]
    n_paper = x_paper.shape[0]

    lp0, lp1 = params['conv'][0], params['conv'][1]
    bd_k0 = [_block_diag(r) for r in lp0['k_rel']]
    bd_v0 = [_block_diag(r) for r in lp0['v_rel']]
    bd_k1 = [_block_diag(r) for r in lp1['k_rel']]
    bd_v1 = [_block_diag(r) for r in lp1['v_rel']]

    x = _hgt_conv(x, ei_list, lp0, bd_k0, bd_v0)
    x = {nt: _ln_gelu(x[nt], params['ln_w'][nt], params['ln_b'][nt]) for nt in NODE_TYPES}
    x = _hgt_conv(x, ei_list, lp1, bd_k1, bd_v1)
    x = {nt: jax.nn.gelu(x[nt]) for nt in NODE_TYPES}

    offsets = {'paper': 0, 'author': n_paper}
    x_hom = jnp.concatenate([x['paper'], x['author']], 0)
    src_list = []
    dst_list = []
    for i, (s_t, rel, d_t) in enumerate(EDGE_TYPES):
        e = ei_list[i]
        src_list.append(e[0] + offsets[s_t])
        dst_list.append(e[1] + offsets[d_t])
    src = jnp.concatenate(src_list)
    dst = jnp.concatenate(dst_list)
    score = _gat_score(x_hom, src, dst, params['gat'])
    vals, perm = jax.lax.top_k(score, POOL)
    pooled = x_hom[perm] * jnp.tanh(vals)[:, None]
    return _mlp_head(pooled.reshape(1, POOL * D), params['mlp'])


# speedup vs baseline: 4.5771x; 1.1132x over previous
import functools
import numpy as np
import jax
import jax.numpy as jnp
from jax import lax
from jax.experimental import pallas as pl
from jax.experimental.pallas import tpu as pltpu
from jax.experimental.pallas import tpu_sc as plsc

NODE_TYPES = ('paper', 'author')
EDGE_TYPES = (('paper', 'cites', 'paper'), ('author', 'writes', 'paper'), ('paper', 'rev_writes', 'author'))
D = 128
HEADS = 8
DH = D // HEADS
POOL = 8
OUT = 16

_BM = 1000


def _mm_kernel(x_ref, w_ref, b_ref, o_ref):
    acc = jnp.dot(x_ref[...], w_ref[...], preferred_element_type=jnp.float32)
    o_ref[...] = acc + b_ref[...]


def _mm(x, w, b):
    M, K = x.shape
    N = w.shape[1]
    bm = _BM if M % _BM == 0 else M
    grid = (M // bm,)
    return pl.pallas_call(
        _mm_kernel,
        grid=grid,
        in_specs=[
            pl.BlockSpec((bm, K), lambda i: (i, 0)),
            pl.BlockSpec((K, N), lambda i: (0, 0)),
            pl.BlockSpec((1, N), lambda i: (0, 0)),
        ],
        out_specs=pl.BlockSpec((bm, N), lambda i: (i, 0)),
        out_shape=jax.ShapeDtypeStruct((M, N), jnp.float32),
    )(x, w, b.reshape(1, N))


def _out_proj_kernel(agg_ref, w_ref, b_ref, res_ref, sk_ref, o_ref):
    a = jax.nn.gelu(agg_ref[...])
    o = jnp.dot(a, w_ref[...], preferred_element_type=jnp.float32) + b_ref[...]
    sk = sk_ref[...]
    o_ref[...] = sk * o + (1.0 - sk) * res_ref[...]


def _out_proj(agg, w, b, res, sk):
    M, K = agg.shape
    N = w.shape[1]
    bm = _BM if M % _BM == 0 else M
    grid = (M // bm,)
    sk_row = jnp.broadcast_to(sk.reshape(1, 1), (1, N))
    return pl.pallas_call(
        _out_proj_kernel,
        grid=grid,
        in_specs=[
            pl.BlockSpec((bm, K), lambda i: (i, 0)),
            pl.BlockSpec((K, N), lambda i: (0, 0)),
            pl.BlockSpec((1, N), lambda i: (0, 0)),
            pl.BlockSpec((bm, N), lambda i: (i, 0)),
            pl.BlockSpec((1, N), lambda i: (0, 0)),
        ],
        out_specs=pl.BlockSpec((bm, N), lambda i: (i, 0)),
        out_shape=jax.ShapeDtypeStruct((M, N), jnp.float32),
    )(agg, w, b.reshape(1, N), res, sk_row)


def _ln_gelu_kernel(x_ref, w_ref, b_ref, o_ref):
    x = x_ref[...]
    mu = x.mean(-1, keepdims=True)
    var = ((x - mu) ** 2).mean(-1, keepdims=True)
    y = (x - mu) / jnp.sqrt(var + 1e-5) * w_ref[...] + b_ref[...]
    o_ref[...] = jax.nn.gelu(y)


def _ln_gelu(x, w, b):
    M, N = x.shape
    bm = _BM if M % _BM == 0 else M
    grid = (M // bm,)
    return pl.pallas_call(
        _ln_gelu_kernel,
        grid=grid,
        in_specs=[
            pl.BlockSpec((bm, N), lambda i: (i, 0)),
            pl.BlockSpec((1, N), lambda i: (0, 0)),
            pl.BlockSpec((1, N), lambda i: (0, 0)),
        ],
        out_specs=pl.BlockSpec((bm, N), lambda i: (i, 0)),
        out_shape=jax.ShapeDtypeStruct((M, N), jnp.float32),
    )(x, w.reshape(1, N), b.reshape(1, N))


def _mlp_kernel(h_ref, w1_ref, b1_ref, w2_ref, b2_ref, w3_ref, b3_ref, w4_ref, b4_ref, o_ref):
    h = h_ref[...]
    h = jax.nn.gelu(jnp.dot(h, w1_ref[...], preferred_element_type=jnp.float32) + b1_ref[...])
    h = jax.nn.gelu(jnp.dot(h, w2_ref[...], preferred_element_type=jnp.float32) + b2_ref[...])
    h = jax.nn.gelu(jnp.dot(h, w3_ref[...], preferred_element_type=jnp.float32) + b3_ref[...])
    h = jnp.dot(h, w4_ref[...], preferred_element_type=jnp.float32) + b4_ref[...]
    o_ref[...] = jnp.nan_to_num(h)


def _mlp_head(h0, mp):
    out = pl.pallas_call(
        _mlp_kernel,
        out_shape=jax.ShapeDtypeStruct((1, OUT), jnp.float32),
    )(h0,
      mp['w1'], mp['b1'].reshape(1, -1),
      mp['w2'], mp['b2'].reshape(1, -1),
      mp['w3'], mp['b3'].reshape(1, -1),
      mp['w4'], mp['b4'].reshape(1, -1))
    return out.reshape(OUT)


_GW = 80
_NW = 32


@functools.partial(jax.jit, static_argnames=('n_edges',))
def _sc_gather3(krel, vrel, qtab, src, dst, n_edges):
    nwin = n_edges // _GW
    mesh = plsc.VectorSubcoreMesh(core_axis_name="c", subcore_axis_name="s")

    @functools.partial(
        pl.kernel, mesh=mesh,
        out_type=(
            jax.ShapeDtypeStruct((n_edges, D), jnp.float32),
            jax.ShapeDtypeStruct((n_edges, D), jnp.float32),
            jax.ShapeDtypeStruct((n_edges, D), jnp.float32),
        ),
        scratch_types=[
            pltpu.VMEM((_GW,), jnp.int32),
            pltpu.VMEM((_GW,), jnp.int32),
            pltpu.VMEM((_GW, D), jnp.float32),
            pltpu.VMEM((_GW, D), jnp.float32),
            pltpu.VMEM((_GW, D), jnp.float32),
            pltpu.SemaphoreType.DMA,
            pltpu.SemaphoreType.DMA,
            pltpu.SemaphoreType.DMA,
        ],
    )
    def k(krel_h, vrel_h, qtab_h, src_h, dst_h, ke_h, ve_h, qe_h,
          idx_s, idx_d, bk, bv, bq, sem1, sem2, sem3):
        wid = lax.axis_index("s") * 2 + lax.axis_index("c")
        trips = (nwin + 31 - wid) // 32

        def win_body(t, carry):
            win = wid + t * _NW
            base = win * _GW
            pltpu.sync_copy(src_h.at[pl.ds(base, _GW)], idx_s)
            pltpu.sync_copy(dst_h.at[pl.ds(base, _GW)], idx_d)
            ck = pltpu.async_copy(krel_h.at[idx_s], bk, sem1)
            cv = pltpu.async_copy(vrel_h.at[idx_s], bv, sem2)
            cq = pltpu.async_copy(qtab_h.at[idx_d], bq, sem3)
            ck.wait()
            cv.wait()
            cq.wait()
            pltpu.sync_copy(bk, ke_h.at[pl.ds(base, _GW)])
            pltpu.sync_copy(bv, ve_h.at[pl.ds(base, _GW)])
            pltpu.sync_copy(bq, qe_h.at[pl.ds(base, _GW)])
            return carry

        lax.fori_loop(0, trips, win_body, 0)

    return k(krel, vrel, qtab, src, dst)


def _segment_softmax(logits, seg, num_segments):
    m = jax.ops.segment_max(logits, seg, num_segments=num_segments)
    m = jnp.where(jnp.isfinite(m), m, 0.0)
    e = jnp.exp(logits - m[seg])
    s = jax.ops.segment_sum(e, seg, num_segments=num_segments)
    return e / (s[seg] + 1e-16)


def _hgt_conv(x_dict, ei_list, lp, bd_k, bd_v):
    k = {}
    q = {}
    v = {}
    for nt in NODE_TYPES:
        kqv = _mm(x_dict[nt], lp['kqv_w'][nt], lp['kqv_b'][nt])
        k[nt] = kqv[:, :D]
        q[nt] = kqv[:, D:2 * D]
        v[nt] = kqv[:, 2 * D:]
    offsets = {}
    off = 0
    for nt in NODE_TYPES:
        offsets[nt] = off
        off += x_dict[nt].shape[0]
    n_total = off
    alphas = []
    msgs = []
    dsts = []
    zero_b = jnp.zeros((D,), jnp.float32)
    for i, (src_t, rel, dst_t) in enumerate(EDGE_TYPES):
        ei = ei_list[i]
        src = ei[0]
        dst = ei[1]
        k_rel = _mm(k[src_t], bd_k[i], zero_b)
        v_rel = _mm(v[src_t], bd_v[i], zero_b)
        k_e, v_e, q_e = _sc_gather3(k_rel, v_rel, q[dst_t], src, dst, n_edges=src.shape[0])
        a = (q_e.reshape(-1, HEADS, DH) * k_e.reshape(-1, HEADS, DH)).sum(-1)
        a = a * lp['p_rel'][i] / np.sqrt(DH)
        alphas.append(a)
        msgs.append(v_e)
        dsts.append(dst + offsets[dst_t])
    alpha = jnp.concatenate(alphas, 0)
    msg = jnp.concatenate(msgs, 0)
    dst_all = jnp.concatenate(dsts, 0)
    attn = _segment_softmax(alpha, dst_all, n_total)
    attn_flat = jnp.repeat(attn, DH, axis=1)
    agg = jax.ops.segment_sum(msg * attn_flat, dst_all, num_segments=n_total)
    out = {}
    for nt in NODE_TYPES:
        o = agg[offsets[nt]:offsets[nt] + x_dict[nt].shape[0]]
        out[nt] = _out_proj(o, lp['out_w'][nt], lp['out_b'][nt], x_dict[nt], jax.nn.sigmoid(lp['skip'][nt]))
    return out


def _block_diag(rel):
    eye = jnp.eye(HEADS, dtype=jnp.float32)
    m = jnp.einsum('hde,hg->hdge', rel, eye)
    return m.reshape(D, D)


def _gat_score(x, src, dst, gp):
    n = x.shape[0]
    loop = jnp.arange(n, dtype=src.dtype)
    s = jnp.concatenate([src, loop])
    t = jnp.concatenate([dst, loop])
    wp = jnp.pad(gp['w'], ((0, 0), (0, 127)))
    h = _mm(x, wp, jnp.zeros((128,), jnp.float32))[:, 0]
    a = jax.nn.leaky_relu(gp['att_src'] * h[s] + gp['att_dst'] * h[t], 0.2)
    attn = _segment_softmax(a, t, n)
    out = jax.ops.segment_sum(h[s] * attn, t, num_segments=n)
    return out + gp['b']


def kernel(x_paper, x_author, ei_cites, ei_weights, ei_rev, params):
    ei_writes = ei_weights
    x = {'paper': x_paper, 'author': x_author}
    ei_list = [ei_cites, ei_writes, ei_rev]
    n_paper = x_paper.shape[0]

    lp0, lp1 = params['conv'][0], params['conv'][1]
    bd_k0 = [_block_diag(r) for r in lp0['k_rel']]
    bd_v0 = [_block_diag(r) for r in lp0['v_rel']]
    bd_k1 = [_block_diag(r) for r in lp1['k_rel']]
    bd_v1 = [_block_diag(r) for r in lp1['v_rel']]

    x = _hgt_conv(x, ei_list, lp0, bd_k0, bd_v0)
    x = {nt: _ln_gelu(x[nt], params['ln_w'][nt], params['ln_b'][nt]) for nt in NODE_TYPES}
    x = _hgt_conv(x, ei_list, lp1, bd_k1, bd_v1)
    x = {nt: jax.nn.gelu(x[nt]) for nt in NODE_TYPES}

    offsets = {'paper': 0, 'author': n_paper}
    x_hom = jnp.concatenate([x['paper'], x['author']], 0)
    src_list = []
    dst_list = []
    for i, (s_t, rel, d_t) in enumerate(EDGE_TYPES):
        e = ei_list[i]
        src_list.append(e[0] + offsets[s_t])
        dst_list.append(e[1] + offsets[d_t])
    src = jnp.concatenate(src_list)
    dst = jnp.concatenate(dst_list)
    score = _gat_score(x_hom, src, dst, params['gat'])
    vals, perm = jax.lax.top_k(score, POOL)
    pooled = x_hom[perm] * jnp.tanh(vals)[:, None]
    return _mlp_head(pooled.reshape(1, POOL * D), params['mlp'])
